# baseline (speedup 1.0000x reference)
import jax
import jax.numpy as jnp
from jax import lax
from jax.experimental import pallas as pl
from jax.experimental.pallas import tpu as pltpu
from jax.experimental.pallas import tpu_sc as plsc

NC = 2
NS = 16
NW = NC * NS

H = 1024
C = 16
CH = C * H
NB = 3
VECS = CH // 16


def _sc_body(x_hbm, pos_hbm, o_hbm, *bufs_and_sems):
    xbufs = bufs_and_sems[0:NB]
    pbufs = bufs_and_sems[NB:2 * NB]
    xsems, psems, osems = bufs_and_sems[2 * NB:]
    n = x_hbm.shape[0]
    pn = pos_hbm.shape[0]
    epw = n // NW
    nch = epw // CH
    wid = lax.axis_index("s") * NC + lax.axis_index("c")
    e0 = wid * epw
    p0 = lax.rem(e0, pn)

    def in_x(g, slot):
        return pltpu.make_async_copy(
            x_hbm.at[pl.ds(e0 + g * CH, CH)], xbufs[slot], xsems.at[slot])

    def in_p(g, slot):
        return pltpu.make_async_copy(
            pos_hbm.at[pl.ds(p0 + g * CH, CH)], pbufs[slot], psems.at[slot])

    def out_x(g, slot):
        return pltpu.make_async_copy(
            xbufs[slot], o_hbm.at[pl.ds(e0 + g * CH, CH)], osems.at[slot])

    def compute(slot):
        xb = xbufs[slot]
        pb = pbufs[slot]

        def body(i, carry):
            sl = pl.ds(i * 16, 16)
            plsc.addupdate(xb.at[sl], pb[sl])
            return carry
        lax.fori_loop(0, VECS, body, 0, unroll=8)

    for g in range(min(NB - 1, nch)):
        in_x(g, g % NB).start()

    for g in range(nch):
        slot = g % NB
        look = g + NB - 1
        if look < nch:
            if g >= 1:
                out_x(g - 1, look % NB).wait()
            in_x(look, look % NB).start()
        in_x(g, slot).wait()
        out_x(g, slot).start()

    for g in range(max(0, nch - NB), nch):
        out_x(g, g % NB).wait()


def sc_add(x1d, pos1d):
    n = x1d.shape[0]
    mesh = plsc.VectorSubcoreMesh(
        core_axis_name="c", subcore_axis_name="s", num_cores=NC, num_subcores=NS)
    kern = pl.kernel(
        _sc_body,
        out_type=jax.ShapeDtypeStruct((n,), jnp.float32),
        mesh=mesh,
        scratch_types=(
            [pltpu.VMEM((CH,), jnp.float32) for _ in range(2 * NB)]
            + [
                pltpu.SemaphoreType.DMA((NB,)),
                pltpu.SemaphoreType.DMA((NB,)),
                pltpu.SemaphoreType.DMA((NB,)),
            ]
        ),
    )
    return kern(x1d, pos1d)


def kernel(x, pos_table):
    batch, seq_len, hidden = x.shape
    out = sc_add(x.reshape(-1), pos_table.reshape(-1))
    return out.reshape(batch, seq_len, hidden)


# speedup vs baseline: 4.0027x; 4.0027x over previous
import jax
import jax.numpy as jnp
from jax.experimental import pallas as pl

BLOCK_S = 512


def _add_kernel(x_ref, pos_ref, o_ref):
    o_ref[...] = x_ref[...] + pos_ref[...]


def kernel(x, pos_table):
    batch, seq_len, hidden = x.shape
    grid = (seq_len // BLOCK_S,)
    return pl.pallas_call(
        _add_kernel,
        grid=grid,
        in_specs=[
            pl.BlockSpec((batch, BLOCK_S, hidden), lambda s: (0, s, 0)),
            pl.BlockSpec((BLOCK_S, hidden), lambda s: (s, 0)),
        ],
        out_specs=pl.BlockSpec((batch, BLOCK_S, hidden), lambda s: (0, s, 0)),
        out_shape=jax.ShapeDtypeStruct(x.shape, x.dtype),
    )(x, pos_table)
